# baseline (speedup 1.0000x reference)
import functools

import jax
import jax.numpy as jnp
from jax import lax
from jax.experimental import pallas as pl
from jax.experimental.pallas import tpu as pltpu
from jax.experimental.pallas import tpu_sc as plsc

NC = 2
NS = 16
NW = NC * NS
CHUNK = 2000
ROWS = 4096
NB = 64

_mesh = plsc.VectorSubcoreMesh(core_axis_name="c", subcore_axis_name="s")
_sc_params = pltpu.CompilerParams(use_tc_tiling_on_sc=False)
_HI = lax.Precision.HIGHEST


def _sc_deg(dst, zeros1, n_pad, epw):
    @functools.partial(
        pl.kernel,
        mesh=_mesh,
        compiler_params=_sc_params,
        out_type=jax.ShapeDtypeStruct((NC, n_pad), jnp.float32),
        scratch_types=[
            pltpu.VMEM((CHUNK,), jnp.int32),
            pltpu.VMEM((CHUNK,), jnp.float32),
            pltpu.VMEM_SHARED((n_pad,), jnp.float32),
        ],
    )
    def deg_kernel(dst_hbm, zeros_hbm, out_hbm, idx_v, ones_v, acc):
        cid = lax.axis_index("c")
        sid = lax.axis_index("s")
        wid = sid * NC + cid
        slc = n_pad // NS

        @pl.loop(0, CHUNK, step=16)
        def _(i):
            ones_v[pl.ds(i, 16)] = jnp.ones((16,), jnp.float32)

        pltpu.sync_copy(zeros_hbm.at[pl.ds(sid * slc, slc)],
                        acc.at[pl.ds(sid * slc, slc)])
        plsc.subcore_barrier()

        base = wid * epw

        @pl.loop(0, epw, step=CHUNK)
        def _(e0):
            pltpu.sync_copy(dst_hbm.at[pl.ds(base + e0, CHUNK)], idx_v)
            pltpu.sync_copy(ones_v, acc.at[idx_v], add=True)

        plsc.subcore_barrier()
        pltpu.sync_copy(acc.at[pl.ds(sid * slc, slc)],
                        out_hbm.at[cid, pl.ds(sid * slc, slc)])

    return deg_kernel(dst, zeros1)


def _sc_agg(g, src, dst, zeros, n_pad, epw, d):
    @functools.partial(
        pl.kernel,
        mesh=_mesh,
        compiler_params=_sc_params,
        out_type=jax.ShapeDtypeStruct((NC, n_pad, d), jnp.float32),
        scratch_types=[
            pltpu.VMEM((CHUNK,), jnp.int32),
            pltpu.VMEM((CHUNK,), jnp.int32),
            pltpu.VMEM((CHUNK, d), jnp.float32),
            pltpu.VMEM_SHARED((n_pad, d), jnp.float32),
        ],
    )
    def agg_kernel(g_hbm, src_hbm, dst_hbm, zeros_hbm, out_hbm,
                   sidx, didx, rows, acc):
        cid = lax.axis_index("c")
        sid = lax.axis_index("s")
        wid = sid * NC + cid
        slc = n_pad // NS

        pltpu.sync_copy(zeros_hbm.at[pl.ds(sid * slc, slc)],
                        acc.at[pl.ds(sid * slc, slc)])
        plsc.subcore_barrier()

        base = wid * epw

        @pl.loop(0, epw, step=CHUNK)
        def _(e0):
            pltpu.sync_copy(src_hbm.at[pl.ds(base + e0, CHUNK)], sidx)
            pltpu.sync_copy(dst_hbm.at[pl.ds(base + e0, CHUNK)], didx)
            pltpu.sync_copy(g_hbm.at[sidx], rows)
            pltpu.sync_copy(rows, acc.at[didx], add=True)

        plsc.subcore_barrier()
        pltpu.sync_copy(acc.at[pl.ds(sid * slc, slc)],
                        out_hbm.at[cid, pl.ds(sid * slc, slc)])

    return agg_kernel(g, src, dst, zeros)


def _sc_agg_fsplit(ga, gb, src, dst, zeros, n_pad, eps, d):
    @functools.partial(
        pl.kernel,
        mesh=_mesh,
        compiler_params=_sc_params,
        out_type=jax.ShapeDtypeStruct((NC, n_pad, d), jnp.float32),
        scratch_types=[
            pltpu.VMEM((CHUNK,), jnp.int32),
            pltpu.VMEM((CHUNK,), jnp.int32),
            pltpu.VMEM((CHUNK, d), jnp.float32),
            pltpu.VMEM_SHARED((n_pad, d), jnp.float32),
        ],
    )
    def agg_kernel(ga_hbm, gb_hbm, src_hbm, dst_hbm, zeros_hbm, out_hbm,
                   sidx, didx, rows, acc):
        cid = lax.axis_index("c")
        sid = lax.axis_index("s")
        slc = n_pad // NS

        pltpu.sync_copy(zeros_hbm.at[pl.ds(sid * slc, slc)],
                        acc.at[pl.ds(sid * slc, slc)])
        plsc.subcore_barrier()

        base = sid * eps

        def run(g_hbm):
            @pl.loop(0, eps, step=CHUNK)
            def _(e0):
                pltpu.sync_copy(src_hbm.at[pl.ds(base + e0, CHUNK)], sidx)
                pltpu.sync_copy(dst_hbm.at[pl.ds(base + e0, CHUNK)], didx)
                pltpu.sync_copy(g_hbm.at[sidx], rows)
                pltpu.sync_copy(rows, acc.at[didx], add=True)

        @pl.when(cid == 0)
        def _():
            run(ga_hbm)

        @pl.when(cid == 1)
        def _():
            run(gb_hbm)

        plsc.subcore_barrier()
        pltpu.sync_copy(acc.at[pl.ds(sid * slc, slc)],
                        out_hbm.at[cid, pl.ds(sid * slc, slc)])

    return agg_kernel(ga, gb, src, dst, zeros)


def _expand(dinv, k, phases):
    outs = []
    ic = lax.broadcasted_iota(jnp.int32, (128, 128), 1)
    ib = lax.broadcasted_iota(jnp.int32, (128, 128), 0)
    step = 128 // k
    for m in range(phases):
        p = (ib == m * step + ic // k).astype(jnp.float32)
        outs.append(jnp.dot(dinv, p, preferred_element_type=jnp.float32,
                            precision=_HI))
    return jnp.stack(outs, axis=1).reshape(32 * phases, 128)


def _tc_prep(degv, x1p, n_pad):
    nblk = n_pad // ROWS

    def body(p0_r, p1_r, x_r, g1_o, r8_o):
        deg = p0_r[...] + p1_r[...] + 1.0
        y = lax.rsqrt(deg)
        dinv = y * (1.5 - 0.5 * deg * y * y)
        r8 = _expand(dinv, 8, 8)
        g1_o[...] = x_r[...] * r8
        r8_o[...] = r8

    return pl.pallas_call(
        body,
        grid=(nblk,),
        in_specs=[
            pl.BlockSpec((32, 128), lambda i: (i, 0)),
            pl.BlockSpec((32, 128), lambda i: (i + 25, 0)),
            pl.BlockSpec((256, 128), lambda i: (i, 0)),
        ],
        out_specs=[
            pl.BlockSpec((256, 128), lambda i: (i, 0)),
            pl.BlockSpec((256, 128), lambda i: (i, 0)),
        ],
        out_shape=[
            jax.ShapeDtypeStruct((n_pad * 8 // 128, 128), jnp.float32),
            jax.ShapeDtypeStruct((n_pad * 8 // 128, 128), jnp.float32),
        ],
    )(degv, degv, x1p)


def _tc_layer1(agg1v, g1, r8, w1bd, b1bd, n_pad):
    nblk = n_pad // ROWS

    def body(a0_r, a1_r, g1_r, r8_r, w_r, b_r, ga_o, gb_o):
        r8v = r8_r[...]
        s = (a0_r[...] + a1_r[...] + g1_r[...]) * r8v
        h = jnp.dot(s, w_r[...], preferred_element_type=jnp.float32,
                    precision=_HI) + b_r[...]
        d2 = jnp.concatenate([r8v, r8v], axis=1)
        g2 = jnp.maximum(h, 0.0) * d2
        ga_o[...] = g2[:, 0:128]
        gb_o[...] = g2[:, 128:256]

    return pl.pallas_call(
        body,
        grid=(nblk,),
        in_specs=[
            pl.BlockSpec((256, 128), lambda i: (i, 0)),
            pl.BlockSpec((256, 128), lambda i: (i + 25, 0)),
            pl.BlockSpec((256, 128), lambda i: (i, 0)),
            pl.BlockSpec((256, 128), lambda i: (i, 0)),
            pl.BlockSpec((128, 256), lambda i: (0, 0)),
            pl.BlockSpec((1, 256), lambda i: (0, 0)),
        ],
        out_specs=[
            pl.BlockSpec((256, 128), lambda i: (i, 0)),
            pl.BlockSpec((256, 128), lambda i: (i, 0)),
        ],
        out_shape=[
            jax.ShapeDtypeStruct((n_pad * 8 // 128, 128), jnp.float32),
            jax.ShapeDtypeStruct((n_pad * 8 // 128, 128), jnp.float32),
        ],
    )(agg1v, agg1v, g1, r8, w1bd, b1bd)


def _tc_layer2_pool(agg2v, g2a, g2b, r8, btT, w2bd, b2bd, Wfc, bfc, n_pad):
    nblk = n_pad // ROWS

    def body(qa_r, qb_r, ga_r, gb_r, r8_r, bt_r, w_r, b_r, wfc_r, bfc_r,
             out_o, sums, counts):
        i = pl.program_id(0)

        @pl.when(i == 0)
        def _():
            sums[...] = jnp.zeros_like(sums)
            counts[...] = jnp.zeros_like(counts)

        s2 = jnp.concatenate([qa_r[...] + ga_r[...],
                              qb_r[...] + gb_r[...]], axis=1)
        d2 = jnp.concatenate([r8_r[...], r8_r[...]], axis=1)
        h2 = jnp.dot(s2 * d2, w_r[...], preferred_element_type=jnp.float32,
                     precision=_HI) + b_r[...]
        h2 = jnp.maximum(h2, 0.0)

        bt = bt_r[...]
        iota_b = lax.broadcasted_iota(jnp.int32, (NB, 256), 0)
        s_acc = sums[...]
        c_acc = counts[...]
        for j in range(4):
            tj = h2[:, 128 * j:128 * (j + 1)]
            for u in range(4):
                phi = 4 * j + u
                m = (bt[phi:phi + 1, :] == iota_b).astype(jnp.float32)
                s_acc = s_acc + jnp.dot(m, tj[:, 32 * u:32 * (u + 1)],
                                        preferred_element_type=jnp.float32,
                                        precision=_HI)
                c_acc = c_acc + jnp.sum(m, axis=1, keepdims=True)
        sums[...] = s_acc
        counts[...] = c_acc

        @pl.when(i == nblk - 1)
        def _():
            pooled = s_acc / jnp.maximum(c_acc, 1.0)
            z = jnp.dot(pooled, wfc_r[...], preferred_element_type=jnp.float32,
                        precision=_HI) + bfc_r[...]
            out_o[...] = jax.nn.sigmoid(z)

    return pl.pallas_call(
        body,
        grid=(nblk,),
        in_specs=[
            pl.BlockSpec((256, 128), lambda i: (i, 0)),
            pl.BlockSpec((256, 128), lambda i: (i + 25, 0)),
            pl.BlockSpec((256, 128), lambda i: (i, 0)),
            pl.BlockSpec((256, 128), lambda i: (i, 0)),
            pl.BlockSpec((256, 128), lambda i: (i, 0)),
            pl.BlockSpec((16, 256), lambda i: (0, i)),
            pl.BlockSpec((256, 512), lambda i: (0, 0)),
            pl.BlockSpec((1, 512), lambda i: (0, 0)),
            pl.BlockSpec((32, 1), lambda i: (0, 0)),
            pl.BlockSpec((1, 1), lambda i: (0, 0)),
        ],
        out_specs=pl.BlockSpec((NB, 1), lambda i: (0, 0)),
        out_shape=jax.ShapeDtypeStruct((NB, 1), jnp.float32),
        scratch_shapes=[
            pltpu.VMEM((NB, 32), jnp.float32),
            pltpu.VMEM((NB, 1), jnp.float32),
        ],
    )(agg2v, agg2v, g2a, g2b, r8, btT, w2bd, b2bd, Wfc, bfc)


def kernel(x, edge_index, batch, W1, b1, W2, b2, Wfc, bfc):
    n = x.shape[0]
    e = edge_index.shape[1]
    n_pad = 102400
    epw = e // NW

    src = edge_index[0]
    dst = edge_index[1]

    x1p = jnp.pad(x, ((0, n_pad - n), (0, 4))).reshape(-1, 128)
    btT = jnp.pad(batch, (0, n_pad - n),
                  constant_values=NB).reshape(n_pad // 16, 16).T

    il = jnp.arange(128)[:, None]
    ic = jnp.arange(256)[None, :]
    half = (ic >= 128).astype(jnp.int32)
    a_out = jnp.where(half == 0, ic // 8, (ic - 128) // 8)
    o_out = jnp.where(half == 0, ic % 8, 8 + (ic - 128) % 8)
    w1p = jnp.pad(W1, ((0, 4), (0, 0)))
    w1bd = jnp.where(il // 8 == a_out, w1p[il % 8, o_out], 0.0)
    b1bd = jnp.concatenate([jnp.tile(b1[:8], 16),
                            jnp.tile(b1[8:], 16)]).reshape(1, 256)

    il2 = jnp.arange(256)[:, None]
    ic2 = jnp.arange(512)[None, :]
    a_in2 = jnp.where(il2 < 128, il2 // 8, (il2 - 128) // 8)
    f_in2 = jnp.where(il2 < 128, il2 % 8, 8 + (il2 - 128) % 8)
    w2bd = jnp.where(a_in2 == ic2 // 32, W2[f_in2, ic2 % 32], 0.0)
    b2bd = jnp.tile(b2, 16).reshape(1, 512)

    z1 = jnp.zeros((n_pad,), jnp.float32)
    z8 = jnp.zeros((n_pad, 8), jnp.float32)

    deg = _sc_deg(dst, z1, n_pad, epw)
    degv = deg.reshape(2 * n_pad // 128, 128)
    g1, r8 = _tc_prep(degv, x1p, n_pad)
    agg1 = _sc_agg(g1.reshape(n_pad, 8), src, dst, z8, n_pad, epw, 8)
    agg1v = agg1.reshape(2 * n_pad * 8 // 128, 128)
    g2a, g2b = _tc_layer1(agg1v, g1, r8, w1bd, b1bd, n_pad)
    agg2 = _sc_agg_fsplit(g2a.reshape(n_pad, 8), g2b.reshape(n_pad, 8),
                          src, dst, z8, n_pad, e // NS, 8)
    agg2v = agg2.reshape(2 * n_pad * 8 // 128, 128)
    out = _tc_layer2_pool(agg2v, g2a, g2b, r8, btT, w2bd, b2bd, Wfc,
                          bfc.reshape(1, 1), n_pad)
    return out


# speedup vs baseline: 1.6277x; 1.6277x over previous
import functools

import jax
import jax.numpy as jnp
from jax import lax
from jax.experimental import pallas as pl
from jax.experimental.pallas import tpu as pltpu
from jax.experimental.pallas import tpu_sc as plsc

NC = 2
NS = 16
NW = NC * NS
CHUNK = 2000
ROWS = 4096
NB = 64

_mesh = plsc.VectorSubcoreMesh(core_axis_name="c", subcore_axis_name="s")
_sc_params = pltpu.CompilerParams(use_tc_tiling_on_sc=False)
_HI = lax.Precision.HIGHEST


def _sc_deg(dst, zeros1, n_pad, epw):
    @functools.partial(
        pl.kernel,
        mesh=_mesh,
        compiler_params=_sc_params,
        out_type=jax.ShapeDtypeStruct((NC, n_pad), jnp.float32),
        scratch_types=[
            pltpu.VMEM((CHUNK,), jnp.int32),
            pltpu.VMEM((CHUNK,), jnp.float32),
            pltpu.VMEM_SHARED((n_pad,), jnp.float32),
        ],
    )
    def deg_kernel(dst_hbm, zeros_hbm, out_hbm, idx_v, ones_v, acc):
        cid = lax.axis_index("c")
        sid = lax.axis_index("s")
        wid = sid * NC + cid
        slc = n_pad // NS

        @pl.loop(0, CHUNK, step=16)
        def _(i):
            ones_v[pl.ds(i, 16)] = jnp.ones((16,), jnp.float32)

        pltpu.sync_copy(zeros_hbm.at[pl.ds(sid * slc, slc)],
                        acc.at[pl.ds(sid * slc, slc)])
        plsc.subcore_barrier()

        base = wid * epw

        @pl.loop(0, epw, step=CHUNK)
        def _(e0):
            pltpu.sync_copy(dst_hbm.at[pl.ds(base + e0, CHUNK)], idx_v)
            pltpu.sync_copy(ones_v, acc.at[idx_v], add=True)

        plsc.subcore_barrier()
        pltpu.sync_copy(acc.at[pl.ds(sid * slc, slc)],
                        out_hbm.at[cid, pl.ds(sid * slc, slc)])

    return deg_kernel(dst, zeros1)


def _sc_agg(g, src, dst, zeros, n_pad, epw, d):
    @functools.partial(
        pl.kernel,
        mesh=_mesh,
        compiler_params=_sc_params,
        out_type=jax.ShapeDtypeStruct((NC, n_pad, d), jnp.float32),
        scratch_types=[
            pltpu.VMEM((CHUNK,), jnp.int32),
            pltpu.VMEM((CHUNK,), jnp.int32),
            pltpu.VMEM((CHUNK, d), jnp.float32),
            pltpu.VMEM_SHARED((n_pad, d), jnp.float32),
        ],
    )
    def agg_kernel(g_hbm, src_hbm, dst_hbm, zeros_hbm, out_hbm,
                   sidx, didx, rows, acc):
        cid = lax.axis_index("c")
        sid = lax.axis_index("s")
        wid = sid * NC + cid
        slc = n_pad // NS

        pltpu.sync_copy(zeros_hbm.at[pl.ds(sid * slc, slc)],
                        acc.at[pl.ds(sid * slc, slc)])
        plsc.subcore_barrier()

        base = wid * epw

        @pl.loop(0, epw, step=CHUNK)
        def _(e0):
            pltpu.sync_copy(src_hbm.at[pl.ds(base + e0, CHUNK)], sidx)
            pltpu.sync_copy(dst_hbm.at[pl.ds(base + e0, CHUNK)], didx)
            pltpu.sync_copy(g_hbm.at[sidx], rows)
            pltpu.sync_copy(rows, acc.at[didx], add=True)

        plsc.subcore_barrier()
        pltpu.sync_copy(acc.at[pl.ds(sid * slc, slc)],
                        out_hbm.at[cid, pl.ds(sid * slc, slc)])

    return agg_kernel(g, src, dst, zeros)


def _sc_agg_fsplit(ga, gb, src, dst, zeros, n_pad, eps, d):
    @functools.partial(
        pl.kernel,
        mesh=_mesh,
        compiler_params=_sc_params,
        out_type=jax.ShapeDtypeStruct((NC, n_pad, d), jnp.float32),
        scratch_types=[
            pltpu.VMEM((CHUNK,), jnp.int32),
            pltpu.VMEM((CHUNK,), jnp.int32),
            pltpu.VMEM((CHUNK, d), jnp.float32),
            pltpu.VMEM_SHARED((n_pad, d), jnp.float32),
        ],
    )
    def agg_kernel(ga_hbm, gb_hbm, src_hbm, dst_hbm, zeros_hbm, out_hbm,
                   sidx, didx, rows, acc):
        cid = lax.axis_index("c")
        sid = lax.axis_index("s")
        slc = n_pad // NS

        pltpu.sync_copy(zeros_hbm.at[pl.ds(sid * slc, slc)],
                        acc.at[pl.ds(sid * slc, slc)])
        plsc.subcore_barrier()

        base = sid * eps

        def run(g_hbm):
            @pl.loop(0, eps, step=CHUNK)
            def _(e0):
                pltpu.sync_copy(src_hbm.at[pl.ds(base + e0, CHUNK)], sidx)
                pltpu.sync_copy(dst_hbm.at[pl.ds(base + e0, CHUNK)], didx)
                pltpu.sync_copy(g_hbm.at[sidx], rows)
                pltpu.sync_copy(rows, acc.at[didx], add=True)

        @pl.when(cid == 0)
        def _():
            run(ga_hbm)

        @pl.when(cid == 1)
        def _():
            run(gb_hbm)

        plsc.subcore_barrier()
        pltpu.sync_copy(acc.at[pl.ds(sid * slc, slc)],
                        out_hbm.at[cid, pl.ds(sid * slc, slc)])

    return agg_kernel(ga, gb, src, dst, zeros)


def _expand(dinv, k, phases):
    outs = []
    ic = lax.broadcasted_iota(jnp.int32, (128, 128), 1)
    ib = lax.broadcasted_iota(jnp.int32, (128, 128), 0)
    step = 128 // k
    for m in range(phases):
        p = (ib == m * step + ic // k).astype(jnp.float32)
        outs.append(jnp.dot(dinv, p, preferred_element_type=jnp.float32,
                            precision=_HI))
    return jnp.stack(outs, axis=1).reshape(32 * phases, 128)


def _tc_prep(degv, x1p, n_pad):
    nblk = n_pad // ROWS

    def body(p0_r, p1_r, x_r, g1_o, r8_o):
        deg = p0_r[...] + p1_r[...] + 1.0
        y = lax.rsqrt(deg)
        dinv = y * (1.5 - 0.5 * deg * y * y)
        r8 = _expand(dinv, 8, 8)
        g1_o[...] = x_r[...] * r8
        r8_o[...] = r8

    return pl.pallas_call(
        body,
        grid=(nblk,),
        in_specs=[
            pl.BlockSpec((32, 128), lambda i: (i, 0)),
            pl.BlockSpec((32, 128), lambda i: (i + 25, 0)),
            pl.BlockSpec((256, 128), lambda i: (i, 0)),
        ],
        out_specs=[
            pl.BlockSpec((256, 128), lambda i: (i, 0)),
            pl.BlockSpec((256, 128), lambda i: (i, 0)),
        ],
        out_shape=[
            jax.ShapeDtypeStruct((n_pad * 8 // 128, 128), jnp.float32),
            jax.ShapeDtypeStruct((n_pad * 8 // 128, 128), jnp.float32),
        ],
    )(degv, degv, x1p)


def _tc_layer1(agg1v, g1, r8, w1bd, b1bd, n_pad):
    nblk = n_pad // ROWS

    def body(a0_r, a1_r, g1_r, r8_r, w_r, b_r, ga_o, gb_o):
        r8v = r8_r[...]
        s = (a0_r[...] + a1_r[...] + g1_r[...]) * r8v
        h = jnp.dot(s, w_r[...], preferred_element_type=jnp.float32,
                    precision=_HI) + b_r[...]
        d2 = jnp.concatenate([r8v, r8v], axis=1)
        g2 = jnp.maximum(h, 0.0) * d2
        ga_o[...] = g2[:, 0:128]
        gb_o[...] = g2[:, 128:256]

    return pl.pallas_call(
        body,
        grid=(nblk,),
        in_specs=[
            pl.BlockSpec((256, 128), lambda i: (i, 0)),
            pl.BlockSpec((256, 128), lambda i: (i + 25, 0)),
            pl.BlockSpec((256, 128), lambda i: (i, 0)),
            pl.BlockSpec((256, 128), lambda i: (i, 0)),
            pl.BlockSpec((128, 256), lambda i: (0, 0)),
            pl.BlockSpec((1, 256), lambda i: (0, 0)),
        ],
        out_specs=[
            pl.BlockSpec((256, 128), lambda i: (i, 0)),
            pl.BlockSpec((256, 128), lambda i: (i, 0)),
        ],
        out_shape=[
            jax.ShapeDtypeStruct((n_pad * 8 // 128, 128), jnp.float32),
            jax.ShapeDtypeStruct((n_pad * 8 // 128, 128), jnp.float32),
        ],
    )(agg1v, agg1v, g1, r8, w1bd, b1bd)


def _tc_layer2_pool(agg2v, g2a, g2b, r8, btT, w2bd, b2bd, Wfc, bfc, n_pad):
    nblk = n_pad // ROWS

    def body(qa_r, qb_r, ga_r, gb_r, r8_r, bt_r, w_r, b_r, wfc_r, bfc_r,
             out_o, sums, counts):
        i = pl.program_id(0)

        @pl.when(i == 0)
        def _():
            sums[...] = jnp.zeros_like(sums)
            counts[...] = jnp.zeros_like(counts)

        s2 = jnp.concatenate([qa_r[...] + ga_r[...],
                              qb_r[...] + gb_r[...]], axis=1)
        d2 = jnp.concatenate([r8_r[...], r8_r[...]], axis=1)
        h2 = jnp.dot(s2 * d2, w_r[...], preferred_element_type=jnp.float32,
                     precision=_HI) + b_r[...]
        h2 = jnp.maximum(h2, 0.0)

        bt = bt_r[...]
        iota_b = lax.broadcasted_iota(jnp.int32, (NB, 256), 0)
        s_acc = sums[...]
        c_acc = counts[...]
        for j in range(4):
            tj = h2[:, 128 * j:128 * (j + 1)]
            for u in range(4):
                phi = 4 * j + u
                m = (bt[phi:phi + 1, :] == iota_b).astype(jnp.float32)
                s_acc = s_acc + jnp.dot(m, tj[:, 32 * u:32 * (u + 1)],
                                        preferred_element_type=jnp.float32,
                                        precision=_HI)
                c_acc = c_acc + jnp.sum(m, axis=1, keepdims=True)
        sums[...] = s_acc
        counts[...] = c_acc

        @pl.when(i == nblk - 1)
        def _():
            pooled = s_acc / jnp.maximum(c_acc, 1.0)
            z = jnp.dot(pooled, wfc_r[...], preferred_element_type=jnp.float32,
                        precision=_HI) + bfc_r[...]
            out_o[...] = jax.nn.sigmoid(z)

    return pl.pallas_call(
        body,
        grid=(nblk,),
        in_specs=[
            pl.BlockSpec((256, 128), lambda i: (i, 0)),
            pl.BlockSpec((256, 128), lambda i: (i + 25, 0)),
            pl.BlockSpec((256, 128), lambda i: (i, 0)),
            pl.BlockSpec((256, 128), lambda i: (i, 0)),
            pl.BlockSpec((256, 128), lambda i: (i, 0)),
            pl.BlockSpec((16, 256), lambda i: (0, i)),
            pl.BlockSpec((256, 512), lambda i: (0, 0)),
            pl.BlockSpec((1, 512), lambda i: (0, 0)),
            pl.BlockSpec((32, 1), lambda i: (0, 0)),
            pl.BlockSpec((1, 1), lambda i: (0, 0)),
        ],
        out_specs=pl.BlockSpec((NB, 1), lambda i: (0, 0)),
        out_shape=jax.ShapeDtypeStruct((NB, 1), jnp.float32),
        scratch_shapes=[
            pltpu.VMEM((NB, 32), jnp.float32),
            pltpu.VMEM((NB, 1), jnp.float32),
        ],
    )(agg2v, agg2v, g2a, g2b, r8, btT, w2bd, b2bd, Wfc, bfc)


def kernel(x, edge_index, batch, W1, b1, W2, b2, Wfc, bfc):
    n = x.shape[0]
    e = edge_index.shape[1]
    n_pad = 102400
    epw = e // NW

    src = edge_index[0]
    dst = edge_index[1]

    x1p = jnp.pad(x, ((0, n_pad - n), (0, 4))).reshape(-1, 128)
    btT = jnp.pad(batch, (0, n_pad - n),
                  constant_values=NB).reshape(n_pad // 16, 16).T

    il = jnp.arange(128)[:, None]
    ic = jnp.arange(256)[None, :]
    a_out = jnp.where(ic < 128, ic // 8, (ic - 128) // 8)
    o_out = jnp.where(ic < 128, ic % 8, 8 + (ic - 128) % 8)
    w1p = jnp.pad(W1, ((0, 4), (0, 0)))
    r1 = (il % 8 == jnp.arange(8)[None, :]).astype(jnp.float32)
    s1 = (jnp.arange(16)[:, None] == o_out).astype(jnp.float32)
    w1bd = (r1 @ w1p @ s1) * (il // 8 == a_out)
    b1bd = jnp.concatenate([jnp.tile(b1[:8], 16),
                            jnp.tile(b1[8:], 16)]).reshape(1, 256)

    il2 = jnp.arange(256)[:, None]
    ic2 = jnp.arange(512)[None, :]
    a_in2 = jnp.where(il2 < 128, il2 // 8, (il2 - 128) // 8)
    f_in2 = jnp.where(il2 < 128, il2 % 8, 8 + (il2 - 128) % 8)
    r2 = (f_in2 == jnp.arange(16)[None, :]).astype(jnp.float32)
    s2 = (jnp.arange(32)[:, None] == ic2 % 32).astype(jnp.float32)
    w2bd = (r2 @ W2 @ s2) * (a_in2 == ic2 // 32)
    b2bd = jnp.tile(b2, 16).reshape(1, 512)

    z1 = jnp.zeros((n_pad,), jnp.float32)
    z8 = jnp.zeros((n_pad, 8), jnp.float32)

    deg = _sc_deg(dst, z1, n_pad, epw)
    degv = deg.reshape(2 * n_pad // 128, 128)
    g1, r8 = _tc_prep(degv, x1p, n_pad)
    agg1 = _sc_agg(g1.reshape(n_pad, 8), src, dst, z8, n_pad, epw, 8)
    agg1v = agg1.reshape(2 * n_pad * 8 // 128, 128)
    g2a, g2b = _tc_layer1(agg1v, g1, r8, w1bd, b1bd, n_pad)
    agg2 = _sc_agg_fsplit(g2a.reshape(n_pad, 8), g2b.reshape(n_pad, 8),
                          src, dst, z8, n_pad, e // NS, 8)
    agg2v = agg2.reshape(2 * n_pad * 8 // 128, 128)
    out = _tc_layer2_pool(agg2v, g2a, g2b, r8, btT, w2bd, b2bd, Wfc,
                          bfc.reshape(1, 1), n_pad)
    return out


# speedup vs baseline: 1.9390x; 1.1913x over previous
import functools

import jax
import jax.numpy as jnp
from jax import lax
from jax.experimental import pallas as pl
from jax.experimental.pallas import tpu as pltpu
from jax.experimental.pallas import tpu_sc as plsc

NC = 2
NS = 16
NW = NC * NS
CHUNK = 5000
ROWS = 4096
NB = 64

_mesh = plsc.VectorSubcoreMesh(core_axis_name="c", subcore_axis_name="s")
_sc_params = pltpu.CompilerParams(use_tc_tiling_on_sc=False)
_HI = lax.Precision.HIGHEST


def _sc_deg(dst, zeros1, n_pad, epw):
    @functools.partial(
        pl.kernel,
        mesh=_mesh,
        compiler_params=_sc_params,
        out_type=jax.ShapeDtypeStruct((NC, n_pad), jnp.float32),
        scratch_types=[
            pltpu.VMEM((CHUNK,), jnp.int32),
            pltpu.VMEM((CHUNK,), jnp.float32),
            pltpu.VMEM_SHARED((n_pad,), jnp.float32),
        ],
    )
    def deg_kernel(dst_hbm, zeros_hbm, out_hbm, idx_v, ones_v, acc):
        cid = lax.axis_index("c")
        sid = lax.axis_index("s")
        wid = sid * NC + cid
        slc = n_pad // NS

        @pl.loop(0, CHUNK, step=16)
        def _(i):
            ones_v[pl.ds(i, 16)] = jnp.ones((16,), jnp.float32)

        pltpu.sync_copy(zeros_hbm.at[pl.ds(sid * slc, slc)],
                        acc.at[pl.ds(sid * slc, slc)])
        plsc.subcore_barrier()

        base = wid * epw

        @pl.loop(0, epw, step=CHUNK)
        def _(e0):
            pltpu.sync_copy(dst_hbm.at[pl.ds(base + e0, CHUNK)], idx_v)
            pltpu.sync_copy(ones_v, acc.at[idx_v], add=True)

        plsc.subcore_barrier()
        pltpu.sync_copy(acc.at[pl.ds(sid * slc, slc)],
                        out_hbm.at[cid, pl.ds(sid * slc, slc)])

    return deg_kernel(dst, zeros1)


def _sc_agg(g, src, dst, zeros, n_pad, epw, d):
    @functools.partial(
        pl.kernel,
        mesh=_mesh,
        compiler_params=_sc_params,
        out_type=jax.ShapeDtypeStruct((NC, n_pad, d), jnp.float32),
        scratch_types=[
            pltpu.VMEM((CHUNK,), jnp.int32),
            pltpu.VMEM((CHUNK,), jnp.int32),
            pltpu.VMEM((CHUNK, d), jnp.float32),
            pltpu.VMEM_SHARED((n_pad, d), jnp.float32),
        ],
    )
    def agg_kernel(g_hbm, src_hbm, dst_hbm, zeros_hbm, out_hbm,
                   sidx, didx, rows, acc):
        cid = lax.axis_index("c")
        sid = lax.axis_index("s")
        wid = sid * NC + cid
        slc = n_pad // NS

        pltpu.sync_copy(zeros_hbm.at[pl.ds(sid * slc, slc)],
                        acc.at[pl.ds(sid * slc, slc)])
        plsc.subcore_barrier()

        base = wid * epw

        @pl.loop(0, epw, step=CHUNK)
        def _(e0):
            pltpu.sync_copy(src_hbm.at[pl.ds(base + e0, CHUNK)], sidx)
            pltpu.sync_copy(dst_hbm.at[pl.ds(base + e0, CHUNK)], didx)
            pltpu.sync_copy(g_hbm.at[sidx], rows)
            pltpu.sync_copy(rows, acc.at[didx], add=True)

        plsc.subcore_barrier()
        pltpu.sync_copy(acc.at[pl.ds(sid * slc, slc)],
                        out_hbm.at[cid, pl.ds(sid * slc, slc)])

    return agg_kernel(g, src, dst, zeros)


def _sc_agg_fsplit(ga, gb, src, dst, zeros, n_pad, eps, d):
    @functools.partial(
        pl.kernel,
        mesh=_mesh,
        compiler_params=_sc_params,
        out_type=jax.ShapeDtypeStruct((NC, n_pad, d), jnp.float32),
        scratch_types=[
            pltpu.VMEM((CHUNK,), jnp.int32),
            pltpu.VMEM((CHUNK,), jnp.int32),
            pltpu.VMEM((CHUNK, d), jnp.float32),
            pltpu.VMEM_SHARED((n_pad, d), jnp.float32),
        ],
    )
    def agg_kernel(ga_hbm, gb_hbm, src_hbm, dst_hbm, zeros_hbm, out_hbm,
                   sidx, didx, rows, acc):
        cid = lax.axis_index("c")
        sid = lax.axis_index("s")
        slc = n_pad // NS

        pltpu.sync_copy(zeros_hbm.at[pl.ds(sid * slc, slc)],
                        acc.at[pl.ds(sid * slc, slc)])
        plsc.subcore_barrier()

        base = sid * eps

        def run(g_hbm):
            @pl.loop(0, eps, step=CHUNK)
            def _(e0):
                pltpu.sync_copy(src_hbm.at[pl.ds(base + e0, CHUNK)], sidx)
                pltpu.sync_copy(dst_hbm.at[pl.ds(base + e0, CHUNK)], didx)
                pltpu.sync_copy(g_hbm.at[sidx], rows)
                pltpu.sync_copy(rows, acc.at[didx], add=True)

        @pl.when(cid == 0)
        def _():
            run(ga_hbm)

        @pl.when(cid == 1)
        def _():
            run(gb_hbm)

        plsc.subcore_barrier()
        pltpu.sync_copy(acc.at[pl.ds(sid * slc, slc)],
                        out_hbm.at[cid, pl.ds(sid * slc, slc)])

    return agg_kernel(ga, gb, src, dst, zeros)


def _expand(dinv, k, phases):
    outs = []
    ic = lax.broadcasted_iota(jnp.int32, (128, 128), 1)
    ib = lax.broadcasted_iota(jnp.int32, (128, 128), 0)
    step = 128 // k
    for m in range(phases):
        p = (ib == m * step + ic // k).astype(jnp.float32)
        outs.append(jnp.dot(dinv, p, preferred_element_type=jnp.float32,
                            precision=_HI))
    return jnp.stack(outs, axis=1).reshape(32 * phases, 128)


def _tc_prep(degv, x1p, n_pad):
    nblk = n_pad // ROWS

    def body(p0_r, p1_r, x_r, g1_o, r8_o):
        deg = p0_r[...] + p1_r[...] + 1.0
        y = lax.rsqrt(deg)
        dinv = y * (1.5 - 0.5 * deg * y * y)
        r8 = _expand(dinv, 8, 8)
        g1_o[...] = x_r[...] * r8
        r8_o[...] = r8

    return pl.pallas_call(
        body,
        grid=(nblk,),
        in_specs=[
            pl.BlockSpec((32, 128), lambda i: (i, 0)),
            pl.BlockSpec((32, 128), lambda i: (i + 25, 0)),
            pl.BlockSpec((256, 128), lambda i: (i, 0)),
        ],
        out_specs=[
            pl.BlockSpec((256, 128), lambda i: (i, 0)),
            pl.BlockSpec((256, 128), lambda i: (i, 0)),
        ],
        out_shape=[
            jax.ShapeDtypeStruct((n_pad * 8 // 128, 128), jnp.float32),
            jax.ShapeDtypeStruct((n_pad * 8 // 128, 128), jnp.float32),
        ],
    )(degv, degv, x1p)


def _tc_layer1(agg1v, g1, r8, w1bd, b1bd, n_pad):
    nblk = n_pad // ROWS

    def body(a0_r, a1_r, g1_r, r8_r, w_r, b_r, ga_o, gb_o):
        r8v = r8_r[...]
        s = (a0_r[...] + a1_r[...] + g1_r[...]) * r8v
        h = jnp.dot(s, w_r[...], preferred_element_type=jnp.float32,
                    precision=_HI) + b_r[...]
        d2 = jnp.concatenate([r8v, r8v], axis=1)
        g2 = jnp.maximum(h, 0.0) * d2
        ga_o[...] = g2[:, 0:128]
        gb_o[...] = g2[:, 128:256]

    return pl.pallas_call(
        body,
        grid=(nblk,),
        in_specs=[
            pl.BlockSpec((256, 128), lambda i: (i, 0)),
            pl.BlockSpec((256, 128), lambda i: (i + 25, 0)),
            pl.BlockSpec((256, 128), lambda i: (i, 0)),
            pl.BlockSpec((256, 128), lambda i: (i, 0)),
            pl.BlockSpec((128, 256), lambda i: (0, 0)),
            pl.BlockSpec((1, 256), lambda i: (0, 0)),
        ],
        out_specs=[
            pl.BlockSpec((256, 128), lambda i: (i, 0)),
            pl.BlockSpec((256, 128), lambda i: (i, 0)),
        ],
        out_shape=[
            jax.ShapeDtypeStruct((n_pad * 8 // 128, 128), jnp.float32),
            jax.ShapeDtypeStruct((n_pad * 8 // 128, 128), jnp.float32),
        ],
    )(agg1v, agg1v, g1, r8, w1bd, b1bd)


def _tc_layer2_pool(agg2v, g2a, g2b, r8, btT, w2bd, b2bd, Wfc, bfc, n_pad):
    nblk = n_pad // ROWS

    def body(qa_r, qb_r, ga_r, gb_r, r8_r, bt_r, w_r, b_r, wfc_r, bfc_r,
             out_o, sums, counts):
        i = pl.program_id(0)

        @pl.when(i == 0)
        def _():
            sums[...] = jnp.zeros_like(sums)
            counts[...] = jnp.zeros_like(counts)

        s2 = jnp.concatenate([qa_r[...] + ga_r[...],
                              qb_r[...] + gb_r[...]], axis=1)
        d2 = jnp.concatenate([r8_r[...], r8_r[...]], axis=1)
        h2 = jnp.dot(s2 * d2, w_r[...], preferred_element_type=jnp.float32,
                     precision=_HI) + b_r[...]
        h2 = jnp.maximum(h2, 0.0)

        bt = bt_r[...]
        iota_b = lax.broadcasted_iota(jnp.int32, (NB, 256), 0)
        s_acc = sums[...]
        c_acc = counts[...]
        for j in range(4):
            tj = h2[:, 128 * j:128 * (j + 1)]
            for u in range(4):
                phi = 4 * j + u
                m = (bt[phi:phi + 1, :] == iota_b).astype(jnp.float32)
                s_acc = s_acc + jnp.dot(m, tj[:, 32 * u:32 * (u + 1)],
                                        preferred_element_type=jnp.float32,
                                        precision=_HI)
                c_acc = c_acc + jnp.sum(m, axis=1, keepdims=True)
        sums[...] = s_acc
        counts[...] = c_acc

        @pl.when(i == nblk - 1)
        def _():
            pooled = s_acc / jnp.maximum(c_acc, 1.0)
            z = jnp.dot(pooled, wfc_r[...], preferred_element_type=jnp.float32,
                        precision=_HI) + bfc_r[...]
            out_o[...] = jax.nn.sigmoid(z)

    return pl.pallas_call(
        body,
        grid=(nblk,),
        in_specs=[
            pl.BlockSpec((256, 128), lambda i: (i, 0)),
            pl.BlockSpec((256, 128), lambda i: (i + 25, 0)),
            pl.BlockSpec((256, 128), lambda i: (i, 0)),
            pl.BlockSpec((256, 128), lambda i: (i, 0)),
            pl.BlockSpec((256, 128), lambda i: (i, 0)),
            pl.BlockSpec((16, 256), lambda i: (0, i)),
            pl.BlockSpec((256, 512), lambda i: (0, 0)),
            pl.BlockSpec((1, 512), lambda i: (0, 0)),
            pl.BlockSpec((32, 1), lambda i: (0, 0)),
            pl.BlockSpec((1, 1), lambda i: (0, 0)),
        ],
        out_specs=pl.BlockSpec((NB, 1), lambda i: (0, 0)),
        out_shape=jax.ShapeDtypeStruct((NB, 1), jnp.float32),
        scratch_shapes=[
            pltpu.VMEM((NB, 32), jnp.float32),
            pltpu.VMEM((NB, 1), jnp.float32),
        ],
    )(agg2v, agg2v, g2a, g2b, r8, btT, w2bd, b2bd, Wfc, bfc)


def kernel(x, edge_index, batch, W1, b1, W2, b2, Wfc, bfc):
    n = x.shape[0]
    e = edge_index.shape[1]
    n_pad = 102400
    epw = e // NW

    src = edge_index[0]
    dst = edge_index[1]

    x1p = jnp.pad(x, ((0, n_pad - n), (0, 4))).reshape(-1, 128)
    btT = jnp.pad(batch, (0, n_pad - n),
                  constant_values=NB).reshape(n_pad // 16, 16).T

    il = jnp.arange(128)[:, None]
    ic = jnp.arange(256)[None, :]
    a_out = jnp.where(ic < 128, ic // 8, (ic - 128) // 8)
    o_out = jnp.where(ic < 128, ic % 8, 8 + (ic - 128) % 8)
    w1p = jnp.pad(W1, ((0, 4), (0, 0)))
    r1 = (il % 8 == jnp.arange(8)[None, :]).astype(jnp.float32)
    s1 = (jnp.arange(16)[:, None] == o_out).astype(jnp.float32)
    w1bd = (r1 @ w1p @ s1) * (il // 8 == a_out)
    b1bd = jnp.concatenate([jnp.tile(b1[:8], 16),
                            jnp.tile(b1[8:], 16)]).reshape(1, 256)

    il2 = jnp.arange(256)[:, None]
    ic2 = jnp.arange(512)[None, :]
    a_in2 = jnp.where(il2 < 128, il2 // 8, (il2 - 128) // 8)
    f_in2 = jnp.where(il2 < 128, il2 % 8, 8 + (il2 - 128) % 8)
    r2 = (f_in2 == jnp.arange(16)[None, :]).astype(jnp.float32)
    s2 = (jnp.arange(32)[:, None] == ic2 % 32).astype(jnp.float32)
    w2bd = (r2 @ W2 @ s2) * (a_in2 == ic2 // 32)
    b2bd = jnp.tile(b2, 16).reshape(1, 512)

    z1 = jnp.zeros((n_pad,), jnp.float32)
    z8 = jnp.zeros((n_pad, 8), jnp.float32)

    deg = _sc_deg(dst, z1, n_pad, epw)
    degv = deg.reshape(2 * n_pad // 128, 128)
    g1, r8 = _tc_prep(degv, x1p, n_pad)
    agg1 = _sc_agg(g1.reshape(n_pad, 8), src, dst, z8, n_pad, epw, 8)
    agg1v = agg1.reshape(2 * n_pad * 8 // 128, 128)
    g2a, g2b = _tc_layer1(agg1v, g1, r8, w1bd, b1bd, n_pad)
    agg2 = _sc_agg_fsplit(g2a.reshape(n_pad, 8), g2b.reshape(n_pad, 8),
                          src, dst, z8, n_pad, e // NS, 8)
    agg2v = agg2.reshape(2 * n_pad * 8 // 128, 128)
    out = _tc_layer2_pool(agg2v, g2a, g2b, r8, btT, w2bd, b2bd, Wfc,
                          bfc.reshape(1, 1), n_pad)
    return out
